# Initial kernel scaffold; baseline (speedup 1.0000x reference)
import jax
import jax.numpy as jnp
from jax.experimental import pallas as pl


def kernel(obs, enc_w, enc_b, emb_weight, dec_w1, dec_b1, dec_w2, dec_b2):
    raise NotImplementedError("write your pallas kernel here")


# speedup vs baseline: 4.3714x; 4.3714x over previous
import jax
import jax.numpy as jnp
from jax.experimental import pallas as pl
from jax.experimental.pallas import tpu as pltpu
from jax.experimental.pallas import tpu_sc as plsc

OBS_DIM = 64
N_CODE_EACH = 512
CODE_DIM = 64
BATCH = 1024
HIDDEN = 256
N_CODE_TOTAL = OBS_DIM * N_CODE_EACH
REP_DIM = OBS_DIM * CODE_DIM

_HIGH = jax.lax.Precision.HIGHEST


def _argmin_body(emb_ref, obsT_ref, encwT_ref, encbT_ref, fidx2_ref, par_ref):
    j = pl.program_id(0)
    Wt = emb_ref[...]
    ohj = jax.lax.broadcasted_iota(jnp.int32, (CODE_DIM, OBS_DIM), 1) == j
    ewc = jnp.sum(jnp.where(ohj, encwT_ref[...], 0.0), axis=1, keepdims=True)
    ebc = jnp.sum(jnp.where(ohj, encbT_ref[...], 0.0), axis=1, keepdims=True)
    ob = obsT_ref[pl.ds(j, 1), :]
    zT = ob * ewc + ebc
    cross = jnp.dot(Wt, zT, preferred_element_type=jnp.float32)
    z2 = jnp.sum(zT * zT, axis=0, keepdims=True)
    w2 = jnp.sum(Wt * Wt, axis=1, keepdims=True)
    dists = (z2 + w2) - 2.0 * cross
    m = jnp.min(dists, axis=0, keepdims=True)
    kio = jax.lax.broadcasted_iota(jnp.int32, dists.shape, 0)
    cand = jnp.where(dists == m, kio, N_CODE_EACH)
    idx = jnp.min(cand, axis=0, keepdims=True)
    fidx2_ref[pl.ds(j, 1), :] = (idx >> 1) + j * (N_CODE_EACH // 2)
    par_ref[pl.ds(j, 1), :] = idx & 1


def _decoder_body(q2_ref, par_ref, obs_ref, encwT_ref, encbT_ref,
                  w1_ref, b1_ref, w2_ref, b2_ref,
                  recon_ref, ze_ref, emb_ref):
    q2 = q2_ref[...]
    sel = jnp.where(par_ref[...] == 0,
                    q2[:, :, :CODE_DIM], q2[:, :, CODE_DIM:])
    emb_ref[...] = jnp.swapaxes(sel, 1, 2)
    qf = sel.reshape(sel.shape[0], REP_DIM)
    h = jnp.dot(qf, w1_ref[...],
                preferred_element_type=jnp.float32) + b1_ref[...]
    h = jnp.maximum(h, 0.0)
    recon_ref[...] = jnp.dot(h, w2_ref[...],
                             preferred_element_type=jnp.float32) + b2_ref[...]
    ze_ref[...] = (obs_ref[...][:, None, :] * encwT_ref[...][None, :, :]
                   + encbT_ref[...][None, :, :])


def _sc_gather(table, fidx2):
    n_idx = fidx2.shape[1]
    window = 128

    @pl.kernel(
        out_type=jax.ShapeDtypeStruct((n_idx, 2 * CODE_DIM), table.dtype),
        mesh=plsc.VectorSubcoreMesh(core_axis_name="core",
                                    subcore_axis_name="subcore"),
    )
    def kern(x_hbm, i_hbm, o_hbm):
        def body(i_vmem, o_vmem):
            pltpu.sync_copy(x_hbm.at[i_vmem.at[0]], o_vmem)

        pltpu.emit_pipeline(
            body,
            grid=(n_idx // window,),
            in_specs=[pl.BlockSpec((1, window), index_map=lambda i: (0, i))],
            out_specs=[pl.BlockSpec((window, 2 * CODE_DIM),
                                    index_map=lambda i: (i, 0))],
            core_axis_name=("core", "subcore"),
            dimension_semantics=(pltpu.PARALLEL,),
        )(i_hbm, o_hbm)

    return kern(table, fidx2)


def kernel(obs, enc_w, enc_b, emb_weight, dec_w1, dec_b1, dec_w2, dec_b2):
    obsT = obs.T
    embT = emb_weight.T

    fidx2T, parT = pl.pallas_call(
        _argmin_body,
        grid=(OBS_DIM,),
        in_specs=[
            pl.BlockSpec((N_CODE_EACH, CODE_DIM), lambda j: (j, 0)),
            pl.BlockSpec((OBS_DIM, BATCH), lambda j: (0, 0)),
            pl.BlockSpec((OBS_DIM, CODE_DIM), lambda j: (0, 0)),
            pl.BlockSpec((OBS_DIM, CODE_DIM), lambda j: (0, 0)),
        ],
        out_specs=[
            pl.BlockSpec((OBS_DIM, BATCH), lambda j: (0, 0)),
            pl.BlockSpec((OBS_DIM, BATCH), lambda j: (0, 0)),
        ],
        out_shape=[
            jax.ShapeDtypeStruct((OBS_DIM, BATCH), jnp.int32),
            jax.ShapeDtypeStruct((OBS_DIM, BATCH), jnp.int32),
        ],
    )(embT, obsT, enc_w.T, enc_b.T)

    fidx2 = fidx2T.T.reshape(1, BATCH * OBS_DIM)
    par3 = parT.T.reshape(BATCH, OBS_DIM, 1)
    pair_table = embT.reshape(N_CODE_TOTAL // 2, 2 * CODE_DIM)
    q2 = _sc_gather(pair_table, fidx2)
    q2v = q2.reshape(BATCH, OBS_DIM, 2 * CODE_DIM)

    bb = 128
    nblk = BATCH // bb
    recon, ze, emb = pl.pallas_call(
        _decoder_body,
        grid=(nblk,),
        in_specs=[
            pl.BlockSpec((bb, OBS_DIM, 2 * CODE_DIM), lambda i: (i, 0, 0)),
            pl.BlockSpec((bb, OBS_DIM, 1), lambda i: (i, 0, 0)),
            pl.BlockSpec((bb, OBS_DIM), lambda i: (i, 0)),
            pl.BlockSpec((CODE_DIM, OBS_DIM), lambda i: (0, 0)),
            pl.BlockSpec((CODE_DIM, OBS_DIM), lambda i: (0, 0)),
            pl.BlockSpec((REP_DIM, HIDDEN), lambda i: (0, 0)),
            pl.BlockSpec((1, HIDDEN), lambda i: (0, 0)),
            pl.BlockSpec((HIDDEN, OBS_DIM), lambda i: (0, 0)),
            pl.BlockSpec((1, OBS_DIM), lambda i: (0, 0)),
        ],
        out_specs=[
            pl.BlockSpec((bb, OBS_DIM), lambda i: (i, 0)),
            pl.BlockSpec((bb, CODE_DIM, OBS_DIM), lambda i: (i, 0, 0)),
            pl.BlockSpec((bb, CODE_DIM, OBS_DIM), lambda i: (i, 0, 0)),
        ],
        out_shape=[
            jax.ShapeDtypeStruct((BATCH, OBS_DIM), jnp.float32),
            jax.ShapeDtypeStruct((BATCH, CODE_DIM, OBS_DIM), jnp.float32),
            jax.ShapeDtypeStruct((BATCH, CODE_DIM, OBS_DIM), jnp.float32),
        ],
    )(q2v, par3, obs, enc_w.T, enc_b.T, dec_w1, dec_b1.reshape(1, HIDDEN),
      dec_w2, dec_b2.reshape(1, OBS_DIM))

    return recon, ze, emb
